# baseline (speedup 1.0000x reference)
import functools

import jax
import jax.numpy as jnp
from jax import lax
from jax.experimental import pallas as pl
from jax.experimental.pallas import tpu as pltpu
from jax.experimental.pallas import tpu_sc as plsc

VOCAB = 1000000
D = 64
NC = 2
NS = 16
NW = NC * NS
L = 16

CH = 128


def _make_gather(n_idx: int):
    per_w = n_idx // NW
    n_chunks = per_w // CH
    mesh = plsc.VectorSubcoreMesh(core_axis_name="c", subcore_axis_name="s")

    def body(table_hbm, idx_hbm, out_hbm, idx_v, rows_v, gsem):
        wid = lax.axis_index("s") * NC + lax.axis_index("c")
        base = wid * per_w
        pltpu.sync_copy(idx_hbm.at[pl.ds(base, per_w)], idx_v)

        def chunk(g, _):
            off = g * CH
            pltpu.async_copy(
                table_hbm.at[idx_v.at[pl.ds(off, CH)]], rows_v, gsem
            ).wait()

            def rstep(r, _):
                iv = plsc.load_gather(
                    idx_v, [jnp.full((L,), off + r, jnp.int32)])
                m = (iv != 0).astype(jnp.float32)
                for q in range(D // L):
                    sl = pl.ds(q * L, L)
                    rows_v[r, sl] = rows_v[r, sl] * m
                return 0

            lax.fori_loop(0, CH, rstep, 0)

            pltpu.sync_copy(rows_v, out_hbm.at[pl.ds(base + off, CH)])
            return 0

        lax.fori_loop(0, n_chunks, chunk, 0)

    return pl.kernel(
        body,
        out_type=jax.ShapeDtypeStruct((n_idx, D), jnp.float32),
        mesh=mesh,
        compiler_params=pltpu.CompilerParams(
            needs_layout_passes=False, use_tc_tiling_on_sc=False),
        scratch_types=[
            pltpu.VMEM((per_w,), jnp.int32),
            pltpu.VMEM((CH, D), jnp.float32),
            pltpu.SemaphoreType.DMA,
        ],
    )


def kernel(price_hist, price_lens, tweet_hist, tweet_lens, embedding_matrix):
    b, h = tweet_hist.shape
    idx = tweet_hist.reshape(-1).astype(jnp.int32)
    out = _make_gather(b * h)(embedding_matrix, idx)
    return out.reshape(b, h, D)


# speedup vs baseline: 1.2158x; 1.2158x over previous
import jax
import jax.numpy as jnp
from jax import lax
from jax.experimental import pallas as pl
from jax.experimental.pallas import tpu as pltpu
from jax.experimental.pallas import tpu_sc as plsc

D = 64
NC = 2
NS = 16
NW = NC * NS
L = 16

CH = 128
NBUF = 8
LA = 4


def _make_gather(n_idx: int):
    per_w = n_idx // NW
    n_ch = per_w // CH
    assert n_ch % NBUF == 0 and n_idx % NW == 0
    mesh = plsc.VectorSubcoreMesh(core_axis_name="c", subcore_axis_name="s")

    def body(table_hbm, idx_hbm, out_hbm, idx_v, rows_v, gsem, osem):
        wid = lax.axis_index("s") * NC + lax.axis_index("c")
        base = wid * per_w
        pltpu.sync_copy(idx_hbm.at[pl.ds(base, per_w)], idx_v)

        def g_copy(g, b):
            return pltpu.make_async_copy(
                table_hbm.at[idx_v.at[pl.ds(g * CH, CH)]],
                rows_v.at[b], gsem.at[b])

        def o_copy(g, b):
            return pltpu.make_async_copy(
                rows_v.at[b], out_hbm.at[pl.ds(base + g * CH, CH)],
                osem.at[b])

        def mask(g, b):
            def rstep(r, _):
                iv = plsc.load_gather(
                    idx_v, [jnp.full((L,), g * CH + r, jnp.int32)])
                m = (iv != 0).astype(jnp.float32)
                for q in range(D // L):
                    sl = pl.ds(q * L, L)
                    rows_v[b, r, sl] = rows_v[b, r, sl] * m
                return 0

            lax.fori_loop(0, CH, rstep, 0)

        for b in range(LA):
            g_copy(b, b).start()

        def outer(t, _):
            for b in range(NBUF):
                g = t * NBUF + b
                gn = g + LA
                bn = (b + LA) % NBUF

                @pl.when(gn < n_ch)
                def _refill():
                    @pl.when(gn >= NBUF)
                    def _wait_prev_out():
                        o_copy(gn - NBUF, bn).wait()

                    g_copy(gn, bn).start()

                g_copy(g, b).wait()
                mask(g, b)
                o_copy(g, b).start()
            return 0

        lax.fori_loop(0, n_ch // NBUF, outer, 0)

        for b in range(NBUF):
            o_copy(n_ch - NBUF + b, b).wait()

    return pl.kernel(
        body,
        out_type=jax.ShapeDtypeStruct((n_idx, D), jnp.float32),
        mesh=mesh,
        compiler_params=pltpu.CompilerParams(
            needs_layout_passes=False, use_tc_tiling_on_sc=False),
        scratch_types=[
            pltpu.VMEM((per_w,), jnp.int32),
            pltpu.VMEM((NBUF, CH, D), jnp.float32),
            pltpu.SemaphoreType.DMA((NBUF,)),
            pltpu.SemaphoreType.DMA((NBUF,)),
        ],
    )


def kernel(price_hist, price_lens, tweet_hist, tweet_lens, embedding_matrix):
    b, h = tweet_hist.shape
    idx = tweet_hist.reshape(-1).astype(jnp.int32)
    out = _make_gather(b * h)(embedding_matrix, idx)
    return out.reshape(b, h, D)


# speedup vs baseline: 1.2802x; 1.0530x over previous
import jax
import jax.numpy as jnp
from jax import lax
from jax.experimental import pallas as pl
from jax.experimental.pallas import tpu as pltpu
from jax.experimental.pallas import tpu_sc as plsc

D = 64
NC = 2
NS = 16
NW = NC * NS
L = 16

CH = 128
NBUF = 8
LA = 4


def _make_gather(n_idx: int):
    per_w = n_idx // NW
    n_ch = per_w // CH
    assert n_ch % NBUF == 0 and n_idx % NW == 0
    mesh = plsc.VectorSubcoreMesh(core_axis_name="c", subcore_axis_name="s")

    def body(table_hbm, idx_hbm, out_hbm, idx_v, rows_v, gsem, osem):
        wid = lax.axis_index("s") * NC + lax.axis_index("c")
        base = wid * per_w
        pltpu.sync_copy(idx_hbm.at[pl.ds(base, per_w)], idx_v)

        def g_copy(g, b):
            return pltpu.make_async_copy(
                table_hbm.at[idx_v.at[pl.ds(g * CH, CH)]],
                rows_v.at[b], gsem.at[b])

        def o_copy(g, b):
            return pltpu.make_async_copy(
                rows_v.at[b], out_hbm.at[pl.ds(base + g * CH, CH)],
                osem.at[b])

        def mask(g, b):
            def rstep(r, _):
                iv = plsc.load_gather(
                    idx_v, [jnp.full((L,), g * CH + r, jnp.int32)])
                m = (iv != 0).astype(jnp.float32)
                for q in range(D // L):
                    sl = pl.ds(q * L, L)
                    rows_v[b, r, sl] = rows_v[b, r, sl] * m
                return 0

            lax.fori_loop(0, CH, rstep, 0)

        for b in range(LA):
            g_copy(b, b).start()

        def chunk_min(g):
            mn = jnp.full((L,), jnp.int32(2**30), jnp.int32)
            for i in range(CH // L):
                mn = jnp.minimum(mn, idx_v[pl.ds(g * CH + i * L, L)])
            return jnp.min(mn)

        def outer(t, _):
            for b in range(NBUF):
                g = t * NBUF + b
                gn = g + LA
                bn = (b + LA) % NBUF

                @pl.when(gn < n_ch)
                def _refill():
                    @pl.when(gn >= NBUF)
                    def _wait_prev_out():
                        o_copy(gn - NBUF, bn).wait()

                    g_copy(gn, bn).start()

                g_copy(g, b).wait()

                @pl.when(chunk_min(g) == 0)
                def _mask():
                    mask(g, b)

                o_copy(g, b).start()
            return 0

        lax.fori_loop(0, n_ch // NBUF, outer, 0)

        for b in range(NBUF):
            o_copy(n_ch - NBUF + b, b).wait()

    return pl.kernel(
        body,
        out_type=jax.ShapeDtypeStruct((n_idx, D), jnp.float32),
        mesh=mesh,
        compiler_params=pltpu.CompilerParams(
            needs_layout_passes=False, use_tc_tiling_on_sc=False),
        scratch_types=[
            pltpu.VMEM((per_w,), jnp.int32),
            pltpu.VMEM((NBUF, CH, D), jnp.float32),
            pltpu.SemaphoreType.DMA((NBUF,)),
            pltpu.SemaphoreType.DMA((NBUF,)),
        ],
    )


def kernel(price_hist, price_lens, tweet_hist, tweet_lens, embedding_matrix):
    b, h = tweet_hist.shape
    idx = tweet_hist.reshape(-1).astype(jnp.int32)
    out = _make_gather(b * h)(embedding_matrix, idx)
    return out.reshape(b, h, D)
